# Initial kernel scaffold; baseline (speedup 1.0000x reference)
import jax
import jax.numpy as jnp
from jax.experimental import pallas as pl


def kernel(coords, features, edge_index, atom_counts, label_binary, W_in, W1, W2, W3, gamma, beta):
    raise NotImplementedError("write your pallas kernel here")


# speedup vs baseline: 5.3488x; 5.3488x over previous
import functools

import jax
import jax.numpy as jnp
from jax import lax
from jax.experimental import pallas as pl
from jax.experimental.pallas import tpu as pltpu
from jax.experimental.pallas import tpu_sc as plsc

N = 100000
E = 1600000
R_RES = 7000
IN_DIM = 30
MAX_ATOMS = 14

NROWS = 100352
TH = 50176
ACC_ROWS = 53248
ASTRIPE = ACC_ROWS // 16
DR = TH // 16
CH = 1024
NCH = 98
TILE_E = CH * NCH
EPAD = 16 * TILE_E
BR = 2048
NB = NROWS // BR
AT = 98048
ATW = AT // 32

_mesh = plsc.VectorSubcoreMesh(core_axis_name="c", subcore_axis_name="s")
_sc_params = pltpu.CompilerParams(use_tc_tiling_on_sc=False)


def _h0_body(x_ref, w_ref, lo_ref, hi_ref):
    h = jnp.maximum(
        jnp.dot(x_ref[...], w_ref[...], preferred_element_type=jnp.float32,
                precision=lax.Precision.HIGHEST), 0.0)
    lo_ref[...] = h[:, :16]
    hi_ref[...] = h[:, 16:]


def _h0(xp, wp):
    return pl.pallas_call(
        _h0_body,
        grid=(NB,),
        in_specs=[
            pl.BlockSpec((BR, 32), lambda i: (i, 0)),
            pl.BlockSpec((32, 32), lambda i: (0, 0)),
        ],
        out_specs=[pl.BlockSpec((BR, 16), lambda i: (i, 0))] * 2,
        out_shape=[jax.ShapeDtypeStruct((NROWS, 16), jnp.float32)] * 2,
    )(xp, wp)


@functools.partial(
    pl.kernel,
    mesh=_mesh,
    out_type=[jax.ShapeDtypeStruct((NROWS, 16), jnp.float32)] * 2,
    scratch_types=[
        pltpu.VMEM((CH,), jnp.int32),
        pltpu.VMEM((CH,), jnp.int32),
        pltpu.VMEM((CH // 128, 128), jnp.int32),
        pltpu.VMEM((CH, 16), jnp.float32),
        pltpu.VMEM((128, 16), jnp.float32),
        pltpu.VMEM((784, 16), jnp.float32),
        pltpu.VMEM_SHARED((ACC_ROWS, 16), jnp.float32),
        pltpu.SemaphoreType.DMA,
    ],
    compiler_params=_sc_params,
)
def _edge_sum(hlo, hhi, srcp, dstp, alo, ahi,
              src_v, dst_v, dst_m, rows_v, zbuf, dbuf, acc_sh, sem):
    cid = lax.axis_index("c")
    t = lax.axis_index("s")

    def zero_zbuf(i, carry):
        zbuf[i, :] = jnp.zeros((16,), jnp.float32)
        return carry

    lax.fori_loop(0, 128, zero_zbuf, 0)

    def one_pass(h_tab, a_out, p):
        def zero_stripe(z, carry):
            pltpu.sync_copy(zbuf, acc_sh.at[pl.ds(t * ASTRIPE + z * 128, 128)])
            return carry

        lax.fori_loop(0, ASTRIPE // 128, zero_stripe, 0)
        plsc.subcore_barrier()

        def body(g, carry):
            c0 = t * NCH + g
            pltpu.sync_copy(srcp.at[pl.ds(c0 * CH, CH)], src_v)
            pltpu.sync_copy(dstp.at[pl.ds(c0 * CH, CH)], dst_v)
            pltpu.async_copy(h_tab.at[src_v], rows_v, sem).wait()
            for j in range(CH // 128):
                for l in range(8):
                    d = dst_v[pl.ds((j * 8 + l) * 16, 16)]
                    garb = TH + jnp.bitwise_and(d, 2047)
                    if p == 0:
                        dm = jnp.where(d < TH, d, garb)
                    else:
                        rel = d - TH
                        dm = jnp.where(rel >= 0, rel, garb)
                    dst_m[j, pl.ds(l * 16, 16)] = dm
            for j in range(CH // 128):
                pltpu.sync_copy(rows_v.at[pl.ds(j * 128, 128)],
                                acc_sh.at[dst_m.at[j]], add=True)
            return carry

        lax.fori_loop(0, NCH, body, 0)
        plsc.subcore_barrier()

        def dbody(d, carry):
            pltpu.sync_copy(acc_sh.at[pl.ds(t * DR + d * 784, 784)], dbuf)
            pltpu.sync_copy(dbuf, a_out.at[pl.ds(p * TH + t * DR + d * 784, 784)])
            return carry

        lax.fori_loop(0, DR // 784, dbody, 0)
        plsc.subcore_barrier()

    def both_passes(h_tab, a_out):
        one_pass(h_tab, a_out, 0)
        one_pass(h_tab, a_out, 1)

    @pl.when(cid == 0)
    def _():
        both_passes(hlo, alo)

    @pl.when(cid == 1)
    def _():
        both_passes(hhi, ahi)


def _round_body(lo_ref, hi_ref, alo_ref, ahi_ref, w_ref, nlo_ref, nhi_ref):
    i = pl.program_id(0)
    z = (jnp.dot(alo_ref[...], w_ref[:16, :], preferred_element_type=jnp.float32,
                 precision=lax.Precision.HIGHEST)
         + jnp.dot(ahi_ref[...], w_ref[16:, :], preferred_element_type=jnp.float32,
                   precision=lax.Precision.HIGHEST))
    rz = jnp.maximum(z, 0.0)
    row = i * BR + lax.broadcasted_iota(jnp.int32, (BR, 16), 0)
    m = row < N
    nlo_ref[...] = jnp.where(m, lo_ref[...] + rz[:, :16], 0.0)
    nhi_ref[...] = jnp.where(m, hi_ref[...] + rz[:, 16:], 0.0)


def _round(hlo, hhi, alo, ahi, w):
    return pl.pallas_call(
        _round_body,
        grid=(NB,),
        in_specs=[pl.BlockSpec((BR, 16), lambda i: (i, 0))] * 4
        + [pl.BlockSpec((32, 32), lambda i: (0, 0))],
        out_specs=[pl.BlockSpec((BR, 16), lambda i: (i, 0))] * 2,
        out_shape=[jax.ShapeDtypeStruct((NROWS, 16), jnp.float32)] * 2,
    )(hlo, hhi, alo, ahi, w)


def _bn_body(lo_ref, hi_ref, g_ref, b_ref, nlo_ref, nhi_ref, acc_ref):
    p = pl.program_id(0)
    j = pl.program_id(1)
    row = j * BR + lax.broadcasted_iota(jnp.int32, (BR, 16), 0)
    m = row < N
    lo = jnp.where(m, lo_ref[...], 0.0)
    hi = jnp.where(m, hi_ref[...], 0.0)

    @pl.when(jnp.logical_and(p == 0, j == 0))
    def _():
        acc_ref[...] = jnp.zeros_like(acc_ref)

    @pl.when(p == 0)
    def _():
        acc_ref[0:1, :] = acc_ref[0:1, :] + jnp.sum(lo, axis=0, keepdims=True)
        acc_ref[1:2, :] = acc_ref[1:2, :] + jnp.sum(hi, axis=0, keepdims=True)
        acc_ref[2:3, :] = acc_ref[2:3, :] + jnp.sum(lo * lo, axis=0, keepdims=True)
        acc_ref[3:4, :] = acc_ref[3:4, :] + jnp.sum(hi * hi, axis=0, keepdims=True)

    @pl.when(p == 1)
    def _():
        inv_n = 1.0 / N
        mean_lo = acc_ref[0:1, :] * inv_n
        mean_hi = acc_ref[1:2, :] * inv_n
        var_lo = acc_ref[2:3, :] * inv_n - mean_lo * mean_lo
        var_hi = acc_ref[3:4, :] * inv_n - mean_hi * mean_hi
        s_lo = g_ref[0:1, :] * lax.rsqrt(var_lo + 1e-5)
        s_hi = g_ref[1:2, :] * lax.rsqrt(var_hi + 1e-5)
        y_lo = jnp.maximum((lo - mean_lo) * s_lo + b_ref[0:1, :], 0.0)
        y_hi = jnp.maximum((hi - mean_hi) * s_hi + b_ref[1:2, :], 0.0)
        nlo_ref[...] = jnp.where(m, y_lo, 0.0)
        nhi_ref[...] = jnp.where(m, y_hi, 0.0)


def _bn(hlo, hhi, g2, b2):
    return pl.pallas_call(
        _bn_body,
        grid=(2, NB),
        in_specs=[pl.BlockSpec((BR, 16), lambda p, j: (j, 0))] * 2
        + [pl.BlockSpec((2, 16), lambda p, j: (0, 0))] * 2,
        out_specs=[pl.BlockSpec((BR, 16), lambda p, j: (j, 0))] * 2,
        out_shape=[jax.ShapeDtypeStruct((NROWS, 16), jnp.float32)] * 2,
        scratch_shapes=[pltpu.VMEM((8, 16), jnp.float32)],
    )(hlo, hhi, g2, b2)


@functools.partial(
    pl.kernel,
    mesh=_mesh,
    out_type=[jax.ShapeDtypeStruct((AT, 16), jnp.float32)] * 2,
    scratch_types=[
        pltpu.VMEM((ATW,), jnp.int32),
        pltpu.VMEM((ATW, 16), jnp.float32),
        pltpu.VMEM((ATW, 16), jnp.float32),
        pltpu.SemaphoreType.DMA,
    ],
    compiler_params=_sc_params,
)
def _final_gather(nlo, nhi, idxg, olo, ohi, idx_v, rlo_v, rhi_v, sem):
    cid = lax.axis_index("c")
    sid = lax.axis_index("s")
    base = (sid * 2 + cid) * ATW
    pltpu.sync_copy(idxg.at[pl.ds(base, ATW)], idx_v)
    pltpu.async_copy(nlo.at[idx_v], rlo_v, sem).wait()
    pltpu.async_copy(nhi.at[idx_v], rhi_v, sem).wait()
    pltpu.sync_copy(rlo_v, olo.at[pl.ds(base, ATW)])
    pltpu.sync_copy(rhi_v, ohi.at[pl.ds(base, ATW)])


def kernel(coords, features, edge_index, atom_counts, label_binary,
           W_in, W1, W2, W3, gamma, beta):
    xp = jnp.pad(features, ((0, NROWS - N), (0, 32 - IN_DIM)))
    wip = jnp.pad(W_in, ((0, 32 - IN_DIM), (0, 0)))
    src = edge_index[0]
    dst = edge_index[1]
    pad_e = EPAD - E
    srcp = jnp.concatenate([src, jnp.zeros((pad_e,), jnp.int32)])
    dstp = jnp.concatenate([dst, jnp.full((pad_e,), N, jnp.int32)])
    g2 = gamma.reshape(2, 16)
    b2 = beta.reshape(2, 16)

    hlo, hhi = _h0(xp, wip)
    for w in (W1, W2, W3):
        alo, ahi = _edge_sum(hlo, hhi, srcp, dstp)
        hlo, hhi = _round(hlo, hhi, alo, ahi, w)
    nlo, nhi = _bn(hlo, hhi, g2, b2)

    counts = atom_counts.astype(jnp.int32)
    offs = jnp.cumsum(counts) - counts
    slot = jnp.arange(MAX_ATOMS, dtype=jnp.int32)
    idx = offs[:, None] + slot[None, :]
    mask = slot[None, :] < counts[:, None]
    idxm = jnp.where(mask, jnp.clip(idx, 0, N - 1), N).reshape(-1)
    idxg = jnp.concatenate(
        [idxm, jnp.full((AT - R_RES * MAX_ATOMS,), N, jnp.int32)])

    olo, ohi = _final_gather(nlo, nhi, idxg)
    nat = R_RES * MAX_ATOMS
    aa = jnp.concatenate([olo[:nat], ohi[:nat]], axis=1).reshape(R_RES, MAX_ATOMS * 32)
    return (aa, label_binary)


# speedup vs baseline: 5.9623x; 1.1147x over previous
import functools

import jax
import jax.numpy as jnp
from jax import lax
from jax.experimental import pallas as pl
from jax.experimental.pallas import tpu as pltpu
from jax.experimental.pallas import tpu_sc as plsc

N = 100000
E = 1600000
R_RES = 7000
IN_DIM = 30
MAX_ATOMS = 14

NROWS = 100352
TH = 50176
ACC_ROWS = 53248
ASTRIPE = ACC_ROWS // 16
DR = TH // 16
CH = 2048
CPR = CH // 128
NCH = 49
TILE_E = CH * NCH
EPAD = 16 * TILE_E
BR = 2048
NB = NROWS // BR
AT = 98304
ATW = AT // 32
AC = 512
ANC = ATW // AC

_mesh = plsc.VectorSubcoreMesh(core_axis_name="c", subcore_axis_name="s")
_sc_params = pltpu.CompilerParams(use_tc_tiling_on_sc=False)


def _h0_body(x_ref, w_ref, lo_ref, hi_ref):
    h = jnp.maximum(
        jnp.dot(x_ref[...], w_ref[...], preferred_element_type=jnp.float32,
                precision=lax.Precision.HIGHEST), 0.0)
    lo_ref[...] = h[:, :16]
    hi_ref[...] = h[:, 16:]


def _h0(xp, wp):
    return pl.pallas_call(
        _h0_body,
        grid=(NB,),
        in_specs=[
            pl.BlockSpec((BR, 32), lambda i: (i, 0)),
            pl.BlockSpec((32, 32), lambda i: (0, 0)),
        ],
        out_specs=[pl.BlockSpec((BR, 16), lambda i: (i, 0))] * 2,
        out_shape=[jax.ShapeDtypeStruct((NROWS, 16), jnp.float32)] * 2,
    )(xp, wp)


@functools.partial(
    pl.kernel,
    mesh=_mesh,
    out_type=[jax.ShapeDtypeStruct((NROWS, 16), jnp.float32)] * 2,
    scratch_types=[
        pltpu.VMEM((CH,), jnp.int32),
        pltpu.VMEM((CPR, 128), jnp.int32),
        pltpu.VMEM((CH, 16), jnp.float32),
        pltpu.VMEM((128, 16), jnp.float32),
        pltpu.VMEM((784, 16), jnp.float32),
        pltpu.VMEM_SHARED((ACC_ROWS, 16), jnp.float32),
        pltpu.SemaphoreType.DMA,
    ],
    compiler_params=_sc_params,
)
def _edge_sum(hlo, hhi, srcp, dlo2, dhi2, alo, ahi,
              src_v, dst_m, rows_v, zbuf, dbuf, acc_sh, sem):
    cid = lax.axis_index("c")
    t = lax.axis_index("s")

    def zero_zbuf(i, carry):
        zbuf[i, :] = jnp.zeros((16,), jnp.float32)
        return carry

    lax.fori_loop(0, 128, zero_zbuf, 0)

    def one_pass(h_tab, d_tab, a_out, p):
        def zero_stripe(z, carry):
            pltpu.sync_copy(zbuf, acc_sh.at[pl.ds(t * ASTRIPE + z * 128, 128)])
            return carry

        lax.fori_loop(0, ASTRIPE // 128, zero_stripe, 0)
        plsc.subcore_barrier()

        def body(g, carry):
            c0 = t * NCH + g
            pltpu.sync_copy(srcp.at[pl.ds(c0 * CH, CH)], src_v)
            pltpu.sync_copy(d_tab.at[pl.ds(c0 * CPR, CPR)], dst_m)
            pltpu.async_copy(h_tab.at[src_v], rows_v, sem).wait()
            for j in range(CPR):
                pltpu.sync_copy(rows_v.at[pl.ds(j * 128, 128)],
                                acc_sh.at[dst_m.at[j]], add=True)
            return carry

        lax.fori_loop(0, NCH, body, 0)
        plsc.subcore_barrier()

        def dbody(d, carry):
            pltpu.sync_copy(acc_sh.at[pl.ds(t * DR + d * 784, 784)], dbuf)
            pltpu.sync_copy(dbuf, a_out.at[pl.ds(p * TH + t * DR + d * 784, 784)])
            return carry

        lax.fori_loop(0, DR // 784, dbody, 0)
        plsc.subcore_barrier()

    def both_passes(h_tab, a_out):
        one_pass(h_tab, dlo2, a_out, 0)
        one_pass(h_tab, dhi2, a_out, 1)

    @pl.when(cid == 0)
    def _():
        both_passes(hlo, alo)

    @pl.when(cid == 1)
    def _():
        both_passes(hhi, ahi)


DW = EPAD // 32 // 128
DC = 49


@functools.partial(
    pl.kernel,
    mesh=_mesh,
    out_type=[jax.ShapeDtypeStruct((EPAD // 128, 128), jnp.int32)] * 2,
    scratch_types=[
        pltpu.VMEM((DC * 128,), jnp.int32),
        pltpu.VMEM((DC, 128), jnp.int32),
        pltpu.VMEM((DC, 128), jnp.int32),
    ],
    compiler_params=_sc_params,
)
def _dmap(dstp, dlo2, dhi2, din, olo_v, ohi_v):
    cid = lax.axis_index("c")
    sid = lax.axis_index("s")
    w = sid * 2 + cid

    def chunk(c, carry):
        r0 = w * DW + c * DC
        pltpu.sync_copy(dstp.at[pl.ds(r0 * 128, DC * 128)], din)

        def row(j, carry2):
            for l in range(8):
                d = din[pl.ds((j * 8 + l) * 16, 16)]
                garb = TH + jnp.bitwise_and(d, 2047)
                olo_v[j, pl.ds(l * 16, 16)] = jnp.where(d < TH, d, garb)
                rel = d - TH
                ohi_v[j, pl.ds(l * 16, 16)] = jnp.where(rel >= 0, rel, garb)
            return carry2

        lax.fori_loop(0, DC, row, 0)
        pltpu.sync_copy(olo_v, dlo2.at[pl.ds(r0, DC)])
        pltpu.sync_copy(ohi_v, dhi2.at[pl.ds(r0, DC)])
        return carry

    lax.fori_loop(0, DW // DC, chunk, 0)


def _round_body(lo_ref, hi_ref, alo_ref, ahi_ref, w_ref, nlo_ref, nhi_ref):
    i = pl.program_id(0)
    z = (jnp.dot(alo_ref[...], w_ref[:16, :], preferred_element_type=jnp.float32,
                 precision=lax.Precision.HIGHEST)
         + jnp.dot(ahi_ref[...], w_ref[16:, :], preferred_element_type=jnp.float32,
                   precision=lax.Precision.HIGHEST))
    rz = jnp.maximum(z, 0.0)
    row = i * BR + lax.broadcasted_iota(jnp.int32, (BR, 16), 0)
    m = row < N
    nlo_ref[...] = jnp.where(m, lo_ref[...] + rz[:, :16], 0.0)
    nhi_ref[...] = jnp.where(m, hi_ref[...] + rz[:, 16:], 0.0)


def _round(hlo, hhi, alo, ahi, w):
    return pl.pallas_call(
        _round_body,
        grid=(NB,),
        in_specs=[pl.BlockSpec((BR, 16), lambda i: (i, 0))] * 4
        + [pl.BlockSpec((32, 32), lambda i: (0, 0))],
        out_specs=[pl.BlockSpec((BR, 16), lambda i: (i, 0))] * 2,
        out_shape=[jax.ShapeDtypeStruct((NROWS, 16), jnp.float32)] * 2,
    )(hlo, hhi, alo, ahi, w)


def _bn_body(lo_ref, hi_ref, g_ref, b_ref, nlo_ref, nhi_ref, acc_ref):
    p = pl.program_id(0)
    j = pl.program_id(1)
    row = j * BR + lax.broadcasted_iota(jnp.int32, (BR, 16), 0)
    m = row < N
    lo = jnp.where(m, lo_ref[...], 0.0)
    hi = jnp.where(m, hi_ref[...], 0.0)

    @pl.when(jnp.logical_and(p == 0, j == 0))
    def _():
        acc_ref[...] = jnp.zeros_like(acc_ref)

    @pl.when(p == 0)
    def _():
        acc_ref[0:1, :] = acc_ref[0:1, :] + jnp.sum(lo, axis=0, keepdims=True)
        acc_ref[1:2, :] = acc_ref[1:2, :] + jnp.sum(hi, axis=0, keepdims=True)
        acc_ref[2:3, :] = acc_ref[2:3, :] + jnp.sum(lo * lo, axis=0, keepdims=True)
        acc_ref[3:4, :] = acc_ref[3:4, :] + jnp.sum(hi * hi, axis=0, keepdims=True)

    @pl.when(p == 1)
    def _():
        inv_n = 1.0 / N
        mean_lo = acc_ref[0:1, :] * inv_n
        mean_hi = acc_ref[1:2, :] * inv_n
        var_lo = acc_ref[2:3, :] * inv_n - mean_lo * mean_lo
        var_hi = acc_ref[3:4, :] * inv_n - mean_hi * mean_hi
        s_lo = g_ref[0:1, :] * lax.rsqrt(var_lo + 1e-5)
        s_hi = g_ref[1:2, :] * lax.rsqrt(var_hi + 1e-5)
        y_lo = jnp.maximum((lo - mean_lo) * s_lo + b_ref[0:1, :], 0.0)
        y_hi = jnp.maximum((hi - mean_hi) * s_hi + b_ref[1:2, :], 0.0)
        nlo_ref[...] = jnp.where(m, y_lo, 0.0)
        nhi_ref[...] = jnp.where(m, y_hi, 0.0)


def _bn(hlo, hhi, g2, b2):
    return pl.pallas_call(
        _bn_body,
        grid=(2, NB),
        in_specs=[pl.BlockSpec((BR, 16), lambda p, j: (j, 0))] * 2
        + [pl.BlockSpec((2, 16), lambda p, j: (0, 0))] * 2,
        out_specs=[pl.BlockSpec((BR, 16), lambda p, j: (j, 0))] * 2,
        out_shape=[jax.ShapeDtypeStruct((NROWS, 16), jnp.float32)] * 2,
        scratch_shapes=[pltpu.VMEM((8, 16), jnp.float32)],
    )(hlo, hhi, g2, b2)


@functools.partial(
    pl.kernel,
    mesh=_mesh,
    out_type=[jax.ShapeDtypeStruct((AT, 16), jnp.float32)] * 2,
    scratch_types=[
        pltpu.VMEM((ATW,), jnp.int32),
        pltpu.VMEM((ATW, 16), jnp.float32),
        pltpu.VMEM((ATW, 16), jnp.float32),
        pltpu.SemaphoreType.DMA,
    ],
    compiler_params=_sc_params,
)
def _final_gather(nlo, nhi, idxg, olo, ohi, idx_v, rlo_v, rhi_v, sem):
    cid = lax.axis_index("c")
    sid = lax.axis_index("s")
    base = (sid * 2 + cid) * ATW
    pltpu.sync_copy(idxg.at[pl.ds(base, ATW)], idx_v)
    pltpu.async_copy(nlo.at[idx_v], rlo_v, sem).wait()
    pltpu.async_copy(nhi.at[idx_v], rhi_v, sem).wait()
    pltpu.sync_copy(rlo_v, olo.at[pl.ds(base, ATW)])
    pltpu.sync_copy(rhi_v, ohi.at[pl.ds(base, ATW)])


def kernel(coords, features, edge_index, atom_counts, label_binary,
           W_in, W1, W2, W3, gamma, beta):
    xp = jnp.pad(features, ((0, NROWS - N), (0, 32 - IN_DIM)))
    wip = jnp.pad(W_in, ((0, 32 - IN_DIM), (0, 0)))
    src = edge_index[0]
    dst = edge_index[1]
    pad_e = EPAD - E
    srcp = jnp.concatenate([src, jnp.zeros((pad_e,), jnp.int32)])
    dstp = jnp.concatenate([dst, jnp.full((pad_e,), N, jnp.int32)])
    dlo2, dhi2 = _dmap(dstp)
    g2 = gamma.reshape(2, 16)
    b2 = beta.reshape(2, 16)

    hlo, hhi = _h0(xp, wip)
    for w in (W1, W2, W3):
        alo, ahi = _edge_sum(hlo, hhi, srcp, dlo2, dhi2)
        hlo, hhi = _round(hlo, hhi, alo, ahi, w)
    nlo, nhi = _bn(hlo, hhi, g2, b2)

    counts = atom_counts.astype(jnp.int32)
    offs = jnp.cumsum(counts) - counts
    slot = jnp.arange(MAX_ATOMS, dtype=jnp.int32)
    idx = offs[:, None] + slot[None, :]
    mask = slot[None, :] < counts[:, None]
    idxm = jnp.where(mask, jnp.clip(idx, 0, N - 1), N).reshape(-1)
    idxg = jnp.concatenate(
        [idxm, jnp.full((AT - R_RES * MAX_ATOMS,), N, jnp.int32)])

    olo, ohi = _final_gather(nlo, nhi, idxg)
    nat = R_RES * MAX_ATOMS
    aa = jnp.concatenate([olo[:nat], ohi[:nat]], axis=1).reshape(R_RES, MAX_ATOMS * 32)
    return (aa, label_binary)


# speedup vs baseline: 6.6879x; 1.1217x over previous
import functools

import jax
import jax.numpy as jnp
from jax import lax
from jax.experimental import pallas as pl
from jax.experimental.pallas import tpu as pltpu
from jax.experimental.pallas import tpu_sc as plsc

N = 100000
E = 1600000
R_RES = 7000
IN_DIM = 30
MAX_ATOMS = 14

NROWS = 100352
TH = 50176
ACC_ROWS = 53248
ASTRIPE = ACC_ROWS // 16
DR = TH // 16
CH = 2048
CPR = CH // 128
NCH = 49
TILE_E = CH * NCH
EPAD = 16 * TILE_E
BR = 2048
NB = NROWS // BR
AT = 98304
ATW = AT // 32
AC = 512
ANC = ATW // AC

_mesh = plsc.VectorSubcoreMesh(core_axis_name="c", subcore_axis_name="s")
_sc_params = pltpu.CompilerParams(use_tc_tiling_on_sc=False)


def _h0_body(x_ref, w_ref, w1_ref, lo_ref, hi_ref, glo_ref, ghi_ref):
    h = jnp.maximum(
        jnp.dot(x_ref[...], w_ref[...], preferred_element_type=jnp.float32), 0.0)
    g = jnp.dot(h, w1_ref[...], preferred_element_type=jnp.float32)
    lo_ref[...] = h[:, :16]
    hi_ref[...] = h[:, 16:]
    glo_ref[...] = g[:, :16]
    ghi_ref[...] = g[:, 16:]


def _h0(xp, wp, w1):
    return pl.pallas_call(
        _h0_body,
        grid=(NB,),
        in_specs=[
            pl.BlockSpec((BR, 32), lambda i: (i, 0)),
            pl.BlockSpec((32, 32), lambda i: (0, 0)),
            pl.BlockSpec((32, 32), lambda i: (0, 0)),
        ],
        out_specs=[pl.BlockSpec((BR, 16), lambda i: (i, 0))] * 4,
        out_shape=[jax.ShapeDtypeStruct((NROWS, 16), jnp.float32)] * 4,
    )(xp, wp, w1)


@functools.partial(
    pl.kernel,
    mesh=_mesh,
    out_type=[jax.ShapeDtypeStruct((NROWS, 16), jnp.float32)] * 2,
    scratch_types=[
        pltpu.VMEM((CH,), jnp.int32),
        pltpu.VMEM((CPR, 128), jnp.int32),
        pltpu.VMEM((CH, 16), jnp.float32),
        pltpu.VMEM_SHARED((ACC_ROWS, 16), jnp.float32),
        pltpu.SemaphoreType.DMA,
        pltpu.SemaphoreType.DMA,
    ],
    compiler_params=_sc_params,
)
def _edge_sum(zrows, hlo, hhi, srcp, dlo2, dhi2, alo, ahi,
              src_v, dst_m, rows_v, acc_sh, sem, sem_s):
    cid = lax.axis_index("c")
    t = lax.axis_index("s")

    def one_pass(h_tab, d_tab, a_out, p):
        pltpu.sync_copy(zrows, acc_sh.at[pl.ds(t * ASTRIPE, ASTRIPE)])
        plsc.subcore_barrier()

        def body(g, carry):
            c0 = t * NCH + g
            pltpu.sync_copy(srcp.at[pl.ds(c0 * CH, CH)], src_v)
            pltpu.sync_copy(d_tab.at[pl.ds(c0 * CPR, CPR)], dst_m)
            pltpu.async_copy(h_tab.at[src_v], rows_v, sem).wait()
            for j in range(CPR):
                pltpu.async_copy(rows_v.at[pl.ds(j * 128, 128)],
                                 acc_sh.at[dst_m.at[j]], sem_s, add=True)
            for j in range(CPR):
                pltpu.make_async_copy(rows_v.at[pl.ds(j * 128, 128)],
                                      acc_sh.at[dst_m.at[j]], sem_s).wait()
            return carry

        lax.fori_loop(0, NCH, body, 0)
        plsc.subcore_barrier()

        pltpu.sync_copy(acc_sh.at[pl.ds(t * DR, DR)],
                        a_out.at[pl.ds(p * TH + t * DR, DR)])
        plsc.subcore_barrier()

    def both_passes(h_tab, a_out):
        one_pass(h_tab, dlo2, a_out, 0)
        one_pass(h_tab, dhi2, a_out, 1)

    @pl.when(cid == 0)
    def _():
        both_passes(hlo, alo)

    @pl.when(cid == 1)
    def _():
        both_passes(hhi, ahi)


DW = EPAD // 32 // 128
DC = 49


@functools.partial(
    pl.kernel,
    mesh=_mesh,
    out_type=[jax.ShapeDtypeStruct((EPAD // 128, 128), jnp.int32)] * 2,
    scratch_types=[
        pltpu.VMEM((DC * 128,), jnp.int32),
        pltpu.VMEM((DC, 128), jnp.int32),
        pltpu.VMEM((DC, 128), jnp.int32),
    ],
    compiler_params=_sc_params,
)
def _dmap(dstp, dlo2, dhi2, din, olo_v, ohi_v):
    cid = lax.axis_index("c")
    sid = lax.axis_index("s")
    w = sid * 2 + cid

    def chunk(c, carry):
        r0 = w * DW + c * DC
        pltpu.sync_copy(dstp.at[pl.ds(r0 * 128, DC * 128)], din)

        def row(j, carry2):
            for l in range(8):
                d = din[pl.ds((j * 8 + l) * 16, 16)]
                garb = TH + jnp.bitwise_and(d, 2047)
                olo_v[j, pl.ds(l * 16, 16)] = jnp.where(d < TH, d, garb)
                rel = d - TH
                ohi_v[j, pl.ds(l * 16, 16)] = jnp.where(rel >= 0, rel, garb)
            return carry2

        lax.fori_loop(0, DC, row, 0)
        pltpu.sync_copy(olo_v, dlo2.at[pl.ds(r0, DC)])
        pltpu.sync_copy(ohi_v, dhi2.at[pl.ds(r0, DC)])
        return carry

    lax.fori_loop(0, DW // DC, chunk, 0)


def _radd_body(lo_ref, hi_ref, alo_ref, ahi_ref, nlo_ref, nhi_ref):
    i = pl.program_id(0)
    row = i * BR + lax.broadcasted_iota(jnp.int32, (BR, 16), 0)
    m = row < N
    nlo_ref[...] = jnp.where(m, lo_ref[...] + jnp.maximum(alo_ref[...], 0.0), 0.0)
    nhi_ref[...] = jnp.where(m, hi_ref[...] + jnp.maximum(ahi_ref[...], 0.0), 0.0)


def _radd(hlo, hhi, alo, ahi):
    return pl.pallas_call(
        _radd_body,
        grid=(NB,),
        in_specs=[pl.BlockSpec((BR, 16), lambda i: (i, 0))] * 4,
        out_specs=[pl.BlockSpec((BR, 16), lambda i: (i, 0))] * 2,
        out_shape=[jax.ShapeDtypeStruct((NROWS, 16), jnp.float32)] * 2,
    )(hlo, hhi, alo, ahi)


def _radd_pre_body(lo_ref, hi_ref, alo_ref, ahi_ref, w_ref,
                   nlo_ref, nhi_ref, glo_ref, ghi_ref):
    i = pl.program_id(0)
    row = i * BR + lax.broadcasted_iota(jnp.int32, (BR, 16), 0)
    m = row < N
    nlo = jnp.where(m, lo_ref[...] + jnp.maximum(alo_ref[...], 0.0), 0.0)
    nhi = jnp.where(m, hi_ref[...] + jnp.maximum(ahi_ref[...], 0.0), 0.0)
    g = (jnp.dot(nlo, w_ref[:16, :], preferred_element_type=jnp.float32)
         + jnp.dot(nhi, w_ref[16:, :], preferred_element_type=jnp.float32))
    nlo_ref[...] = nlo
    nhi_ref[...] = nhi
    glo_ref[...] = g[:, :16]
    ghi_ref[...] = g[:, 16:]


def _radd_pre(hlo, hhi, alo, ahi, w):
    return pl.pallas_call(
        _radd_pre_body,
        grid=(NB,),
        in_specs=[pl.BlockSpec((BR, 16), lambda i: (i, 0))] * 4
        + [pl.BlockSpec((32, 32), lambda i: (0, 0))],
        out_specs=[pl.BlockSpec((BR, 16), lambda i: (i, 0))] * 4,
        out_shape=[jax.ShapeDtypeStruct((NROWS, 16), jnp.float32)] * 4,
    )(hlo, hhi, alo, ahi, w)


def _bn_body(lo_ref, hi_ref, g_ref, b_ref, nlo_ref, nhi_ref, acc_ref):
    p = pl.program_id(0)
    j = pl.program_id(1)
    row = j * BR + lax.broadcasted_iota(jnp.int32, (BR, 16), 0)
    m = row < N
    lo = jnp.where(m, lo_ref[...], 0.0)
    hi = jnp.where(m, hi_ref[...], 0.0)

    @pl.when(jnp.logical_and(p == 0, j == 0))
    def _():
        acc_ref[...] = jnp.zeros_like(acc_ref)

    @pl.when(p == 0)
    def _():
        acc_ref[0:1, :] = acc_ref[0:1, :] + jnp.sum(lo, axis=0, keepdims=True)
        acc_ref[1:2, :] = acc_ref[1:2, :] + jnp.sum(hi, axis=0, keepdims=True)
        acc_ref[2:3, :] = acc_ref[2:3, :] + jnp.sum(lo * lo, axis=0, keepdims=True)
        acc_ref[3:4, :] = acc_ref[3:4, :] + jnp.sum(hi * hi, axis=0, keepdims=True)

    @pl.when(p == 1)
    def _():
        inv_n = 1.0 / N
        mean_lo = acc_ref[0:1, :] * inv_n
        mean_hi = acc_ref[1:2, :] * inv_n
        var_lo = acc_ref[2:3, :] * inv_n - mean_lo * mean_lo
        var_hi = acc_ref[3:4, :] * inv_n - mean_hi * mean_hi
        s_lo = g_ref[0:1, :] * lax.rsqrt(var_lo + 1e-5)
        s_hi = g_ref[1:2, :] * lax.rsqrt(var_hi + 1e-5)
        y_lo = jnp.maximum((lo - mean_lo) * s_lo + b_ref[0:1, :], 0.0)
        y_hi = jnp.maximum((hi - mean_hi) * s_hi + b_ref[1:2, :], 0.0)
        nlo_ref[...] = jnp.where(m, y_lo, 0.0)
        nhi_ref[...] = jnp.where(m, y_hi, 0.0)


def _bn(hlo, hhi, g2, b2):
    return pl.pallas_call(
        _bn_body,
        grid=(2, NB),
        in_specs=[pl.BlockSpec((BR, 16), lambda p, j: (j, 0))] * 2
        + [pl.BlockSpec((2, 16), lambda p, j: (0, 0))] * 2,
        out_specs=[pl.BlockSpec((BR, 16), lambda p, j: (j, 0))] * 2,
        out_shape=[jax.ShapeDtypeStruct((NROWS, 16), jnp.float32)] * 2,
        scratch_shapes=[pltpu.VMEM((8, 16), jnp.float32)],
    )(hlo, hhi, g2, b2)


@functools.partial(
    pl.kernel,
    mesh=_mesh,
    out_type=[jax.ShapeDtypeStruct((AT, 16), jnp.float32)] * 2,
    scratch_types=[
        pltpu.VMEM((ATW,), jnp.int32),
        pltpu.VMEM((2, AC, 16), jnp.float32),
        pltpu.VMEM((2, AC, 16), jnp.float32),
        pltpu.SemaphoreType.DMA,
        pltpu.SemaphoreType.DMA,
    ],
    compiler_params=_sc_params,
)
def _final_gather(nlo, nhi, idxg, olo, ohi, idx_v, rlo_v, rhi_v, s0, s1):
    cid = lax.axis_index("c")
    sid = lax.axis_index("s")
    sems = (s0, s1)
    base = (sid * 2 + cid) * ATW
    pltpu.sync_copy(idxg.at[pl.ds(base, ATW)], idx_v)

    def gi(c, b):
        pltpu.async_copy(nlo.at[idx_v.at[pl.ds(c * AC, AC)]], rlo_v.at[b],
                         sems[b])
        pltpu.async_copy(nhi.at[idx_v.at[pl.ds(c * AC, AC)]], rhi_v.at[b],
                         sems[b])

    def gw(c, b):
        pltpu.make_async_copy(nlo.at[idx_v.at[pl.ds(c * AC, AC)]],
                              rlo_v.at[b], sems[b]).wait()
        pltpu.make_async_copy(nhi.at[idx_v.at[pl.ds(c * AC, AC)]],
                              rhi_v.at[b], sems[b]).wait()

    gi(0, 0)
    for c in range(ANC):
        b = c % 2
        if c + 1 < ANC:
            gi(c + 1, 1 - b)
        gw(c, b)
        pltpu.sync_copy(rlo_v.at[b], olo.at[pl.ds(base + c * AC, AC)])
        pltpu.sync_copy(rhi_v.at[b], ohi.at[pl.ds(base + c * AC, AC)])


def kernel(coords, features, edge_index, atom_counts, label_binary,
           W_in, W1, W2, W3, gamma, beta):
    xp = jnp.pad(features, ((0, NROWS - N), (0, 32 - IN_DIM)))
    wip = jnp.pad(W_in, ((0, 32 - IN_DIM), (0, 0)))
    src = edge_index[0]
    dst = edge_index[1]
    pad_e = EPAD - E
    srcp = jnp.concatenate([src, jnp.zeros((pad_e,), jnp.int32)])
    dstp = jnp.concatenate([dst, jnp.full((pad_e,), N, jnp.int32)])
    dlo2, dhi2 = _dmap(dstp)
    g2 = gamma.reshape(2, 16)
    b2 = beta.reshape(2, 16)

    zrows = jnp.zeros((ASTRIPE, 16), jnp.float32)
    hlo, hhi, glo, ghi = _h0(xp, wip, W1)
    for wn in (W2, W3, None):
        alo, ahi = _edge_sum(zrows, glo, ghi, srcp, dlo2, dhi2)
        if wn is None:
            hlo, hhi = _radd(hlo, hhi, alo, ahi)
        else:
            hlo, hhi, glo, ghi = _radd_pre(hlo, hhi, alo, ahi, wn)
    nlo, nhi = _bn(hlo, hhi, g2, b2)

    counts = atom_counts.astype(jnp.int32)
    offs = jnp.cumsum(counts) - counts
    slot = jnp.arange(MAX_ATOMS, dtype=jnp.int32)
    idx = offs[:, None] + slot[None, :]
    mask = slot[None, :] < counts[:, None]
    idxm = jnp.where(mask, jnp.clip(idx, 0, N - 1), N).reshape(-1)
    idxg = jnp.concatenate(
        [idxm, jnp.full((AT - R_RES * MAX_ATOMS,), N, jnp.int32)])

    olo, ohi = _final_gather(nlo, nhi, idxg)
    nat = R_RES * MAX_ATOMS
    aa = jnp.concatenate([olo[:nat], ohi[:nat]], axis=1).reshape(R_RES, MAX_ATOMS * 32)
    return (aa, label_binary)


# speedup vs baseline: 8.2206x; 1.2292x over previous
import functools

import jax
import jax.numpy as jnp
from jax import lax
from jax.experimental import pallas as pl
from jax.experimental.pallas import tpu as pltpu
from jax.experimental.pallas import tpu_sc as plsc

N = 100000
E = 1600000
R_RES = 7000
IN_DIM = 30
MAX_ATOMS = 14

NROWS = 100352
TH = 50176
ACC_ROWS = 53248
ASTRIPE = ACC_ROWS // 16
DR = TH // 16
CH = 2048
CPR = CH // 128
NCH = 49
TILE_E = CH * NCH
EPAD = 16 * TILE_E
BR = 2048
NB = NROWS // BR
AT = 98304
ATW = AT // 32
AC = 512
ANC = ATW // AC

_mesh = plsc.VectorSubcoreMesh(core_axis_name="c", subcore_axis_name="s")
_sc_params = pltpu.CompilerParams(use_tc_tiling_on_sc=False)


def _h0_body(x_ref, w_ref, w1_ref, lo_ref, hi_ref, glo_ref, ghi_ref):
    h = jnp.maximum(
        jnp.dot(x_ref[...], w_ref[...], preferred_element_type=jnp.float32), 0.0)
    g = jnp.dot(h, w1_ref[...], preferred_element_type=jnp.float32)
    lo_ref[...] = h[:, :16]
    hi_ref[...] = h[:, 16:]
    glo_ref[...] = g[:, :16]
    ghi_ref[...] = g[:, 16:]


def _h0(xp, wp, w1):
    return pl.pallas_call(
        _h0_body,
        grid=(NB,),
        in_specs=[
            pl.BlockSpec((BR, 32), lambda i: (i, 0)),
            pl.BlockSpec((32, 32), lambda i: (0, 0)),
            pl.BlockSpec((32, 32), lambda i: (0, 0)),
        ],
        out_specs=[pl.BlockSpec((BR, 16), lambda i: (i, 0))] * 4,
        out_shape=[jax.ShapeDtypeStruct((NROWS, 16), jnp.float32)] * 4,
    )(xp, wp, w1)


@functools.partial(
    pl.kernel,
    mesh=_mesh,
    out_type=[jax.ShapeDtypeStruct((NROWS, 16), jnp.float32)] * 2,
    scratch_types=[
        pltpu.VMEM((2, CH), jnp.int32),
        pltpu.VMEM((2, CPR, 128), jnp.int32),
        pltpu.VMEM((2, CH, 16), jnp.float32),
        pltpu.VMEM_SHARED((ACC_ROWS, 16), jnp.float32),
        pltpu.SemaphoreType.DMA,
        pltpu.SemaphoreType.DMA,
        pltpu.SemaphoreType.DMA,
    ],
    compiler_params=_sc_params,
)
def _edge_sum(zrows, hlo, hhi, srcp, dlo2, dhi2, alo, ahi,
              src_v, dst_m, rows_v, acc_sh, sem_g0, sem_g1, sem_s):
    cid = lax.axis_index("c")
    t = lax.axis_index("s")
    sem_g = (sem_g0, sem_g1)

    def one_pass(h_tab, d_tab, a_out, p):
        pltpu.sync_copy(zrows, acc_sh.at[pl.ds(t * ASTRIPE, ASTRIPE)])
        plsc.subcore_barrier()

        def idx_load(c, b):
            c0 = t * NCH + c
            pltpu.sync_copy(srcp.at[pl.ds(c0 * CH, CH)], src_v.at[b])
            pltpu.sync_copy(d_tab.at[pl.ds(c0 * CPR, CPR)], dst_m.at[b])

        def g_issue(b):
            pltpu.async_copy(h_tab.at[src_v.at[b]], rows_v.at[b], sem_g[b])

        def g_wait(b):
            pltpu.make_async_copy(h_tab.at[src_v.at[b]], rows_v.at[b],
                                  sem_g[b]).wait()

        def scat(b):
            for j in range(CPR):
                pltpu.async_copy(rows_v.at[b].at[pl.ds(j * 128, 128)],
                                 acc_sh.at[dst_m.at[b].at[j]], sem_s, add=True)
            for j in range(CPR):
                pltpu.make_async_copy(rows_v.at[b].at[pl.ds(j * 128, 128)],
                                      acc_sh.at[dst_m.at[b].at[j]], sem_s).wait()

        idx_load(0, 0)
        g_issue(0)

        def body(g2, carry):
            for k in (0, 1):
                c = 2 * g2 + k
                b, b1 = k, 1 - k

                @pl.when(c < NCH)
                def _():
                    @pl.when(c + 1 < NCH)
                    def _():
                        idx_load(c + 1, b1)

                    g_wait(b)

                    @pl.when(c + 1 < NCH)
                    def _():
                        g_issue(b1)

                    scat(b)

            return carry

        lax.fori_loop(0, (NCH + 1) // 2, body, 0)
        plsc.subcore_barrier()

        pltpu.sync_copy(acc_sh.at[pl.ds(t * DR, DR)],
                        a_out.at[pl.ds(p * TH + t * DR, DR)])
        plsc.subcore_barrier()

    def both_passes(h_tab, a_out):
        one_pass(h_tab, dlo2, a_out, 0)
        one_pass(h_tab, dhi2, a_out, 1)

    @pl.when(cid == 0)
    def _():
        both_passes(hlo, alo)

    @pl.when(cid == 1)
    def _():
        both_passes(hhi, ahi)


DW = EPAD // 32 // 128
DC = 49


@functools.partial(
    pl.kernel,
    mesh=_mesh,
    out_type=[jax.ShapeDtypeStruct((EPAD // 128, 128), jnp.int32)] * 2,
    scratch_types=[
        pltpu.VMEM((DC * 128,), jnp.int32),
        pltpu.VMEM((DC, 128), jnp.int32),
        pltpu.VMEM((DC, 128), jnp.int32),
    ],
    compiler_params=_sc_params,
)
def _dmap(dstp, dlo2, dhi2, din, olo_v, ohi_v):
    cid = lax.axis_index("c")
    sid = lax.axis_index("s")
    w = sid * 2 + cid

    def chunk(c, carry):
        r0 = w * DW + c * DC
        pltpu.sync_copy(dstp.at[pl.ds(r0 * 128, DC * 128)], din)

        def row(j, carry2):
            for l in range(8):
                d = din[pl.ds((j * 8 + l) * 16, 16)]
                garb = TH + jnp.bitwise_and(d, 2047)
                olo_v[j, pl.ds(l * 16, 16)] = jnp.where(d < TH, d, garb)
                rel = d - TH
                ohi_v[j, pl.ds(l * 16, 16)] = jnp.where(rel >= 0, rel, garb)
            return carry2

        lax.fori_loop(0, DC, row, 0)
        pltpu.sync_copy(olo_v, dlo2.at[pl.ds(r0, DC)])
        pltpu.sync_copy(ohi_v, dhi2.at[pl.ds(r0, DC)])
        return carry

    lax.fori_loop(0, DW // DC, chunk, 0)


def _radd_body(lo_ref, hi_ref, alo_ref, ahi_ref, nlo_ref, nhi_ref):
    i = pl.program_id(0)
    row = i * BR + lax.broadcasted_iota(jnp.int32, (BR, 16), 0)
    m = row < N
    nlo_ref[...] = jnp.where(m, lo_ref[...] + jnp.maximum(alo_ref[...], 0.0), 0.0)
    nhi_ref[...] = jnp.where(m, hi_ref[...] + jnp.maximum(ahi_ref[...], 0.0), 0.0)


def _radd(hlo, hhi, alo, ahi):
    return pl.pallas_call(
        _radd_body,
        grid=(NB,),
        in_specs=[pl.BlockSpec((BR, 16), lambda i: (i, 0))] * 4,
        out_specs=[pl.BlockSpec((BR, 16), lambda i: (i, 0))] * 2,
        out_shape=[jax.ShapeDtypeStruct((NROWS, 16), jnp.float32)] * 2,
    )(hlo, hhi, alo, ahi)


def _radd_pre_body(lo_ref, hi_ref, alo_ref, ahi_ref, w_ref,
                   nlo_ref, nhi_ref, glo_ref, ghi_ref):
    i = pl.program_id(0)
    row = i * BR + lax.broadcasted_iota(jnp.int32, (BR, 16), 0)
    m = row < N
    nlo = jnp.where(m, lo_ref[...] + jnp.maximum(alo_ref[...], 0.0), 0.0)
    nhi = jnp.where(m, hi_ref[...] + jnp.maximum(ahi_ref[...], 0.0), 0.0)
    g = (jnp.dot(nlo, w_ref[:16, :], preferred_element_type=jnp.float32)
         + jnp.dot(nhi, w_ref[16:, :], preferred_element_type=jnp.float32))
    nlo_ref[...] = nlo
    nhi_ref[...] = nhi
    glo_ref[...] = g[:, :16]
    ghi_ref[...] = g[:, 16:]


def _radd_pre(hlo, hhi, alo, ahi, w):
    return pl.pallas_call(
        _radd_pre_body,
        grid=(NB,),
        in_specs=[pl.BlockSpec((BR, 16), lambda i: (i, 0))] * 4
        + [pl.BlockSpec((32, 32), lambda i: (0, 0))],
        out_specs=[pl.BlockSpec((BR, 16), lambda i: (i, 0))] * 4,
        out_shape=[jax.ShapeDtypeStruct((NROWS, 16), jnp.float32)] * 4,
    )(hlo, hhi, alo, ahi, w)


def _bn_body(lo_ref, hi_ref, g_ref, b_ref, nlo_ref, nhi_ref, acc_ref):
    p = pl.program_id(0)
    j = pl.program_id(1)
    row = j * BR + lax.broadcasted_iota(jnp.int32, (BR, 16), 0)
    m = row < N
    lo = jnp.where(m, lo_ref[...], 0.0)
    hi = jnp.where(m, hi_ref[...], 0.0)

    @pl.when(jnp.logical_and(p == 0, j == 0))
    def _():
        acc_ref[...] = jnp.zeros_like(acc_ref)

    @pl.when(p == 0)
    def _():
        acc_ref[0:1, :] = acc_ref[0:1, :] + jnp.sum(lo, axis=0, keepdims=True)
        acc_ref[1:2, :] = acc_ref[1:2, :] + jnp.sum(hi, axis=0, keepdims=True)
        acc_ref[2:3, :] = acc_ref[2:3, :] + jnp.sum(lo * lo, axis=0, keepdims=True)
        acc_ref[3:4, :] = acc_ref[3:4, :] + jnp.sum(hi * hi, axis=0, keepdims=True)

    @pl.when(p == 1)
    def _():
        inv_n = 1.0 / N
        mean_lo = acc_ref[0:1, :] * inv_n
        mean_hi = acc_ref[1:2, :] * inv_n
        var_lo = acc_ref[2:3, :] * inv_n - mean_lo * mean_lo
        var_hi = acc_ref[3:4, :] * inv_n - mean_hi * mean_hi
        s_lo = g_ref[0:1, :] * lax.rsqrt(var_lo + 1e-5)
        s_hi = g_ref[1:2, :] * lax.rsqrt(var_hi + 1e-5)
        y_lo = jnp.maximum((lo - mean_lo) * s_lo + b_ref[0:1, :], 0.0)
        y_hi = jnp.maximum((hi - mean_hi) * s_hi + b_ref[1:2, :], 0.0)
        nlo_ref[...] = jnp.where(m, y_lo, 0.0)
        nhi_ref[...] = jnp.where(m, y_hi, 0.0)


def _bn(hlo, hhi, g2, b2):
    return pl.pallas_call(
        _bn_body,
        grid=(2, NB),
        in_specs=[pl.BlockSpec((BR, 16), lambda p, j: (j, 0))] * 2
        + [pl.BlockSpec((2, 16), lambda p, j: (0, 0))] * 2,
        out_specs=[pl.BlockSpec((BR, 16), lambda p, j: (j, 0))] * 2,
        out_shape=[jax.ShapeDtypeStruct((NROWS, 16), jnp.float32)] * 2,
        scratch_shapes=[pltpu.VMEM((8, 16), jnp.float32)],
    )(hlo, hhi, g2, b2)


@functools.partial(
    pl.kernel,
    mesh=_mesh,
    out_type=[jax.ShapeDtypeStruct((AT, 16), jnp.float32)] * 2,
    scratch_types=[
        pltpu.VMEM((ATW,), jnp.int32),
        pltpu.VMEM((2, AC, 16), jnp.float32),
        pltpu.VMEM((2, AC, 16), jnp.float32),
        pltpu.SemaphoreType.DMA,
        pltpu.SemaphoreType.DMA,
    ],
    compiler_params=_sc_params,
)
def _final_gather(nlo, nhi, idxg, olo, ohi, idx_v, rlo_v, rhi_v, s0, s1):
    cid = lax.axis_index("c")
    sid = lax.axis_index("s")
    sems = (s0, s1)
    base = (sid * 2 + cid) * ATW
    pltpu.sync_copy(idxg.at[pl.ds(base, ATW)], idx_v)

    def gi(c, b):
        pltpu.async_copy(nlo.at[idx_v.at[pl.ds(c * AC, AC)]], rlo_v.at[b],
                         sems[b])
        pltpu.async_copy(nhi.at[idx_v.at[pl.ds(c * AC, AC)]], rhi_v.at[b],
                         sems[b])

    def gw(c, b):
        pltpu.make_async_copy(nlo.at[idx_v.at[pl.ds(c * AC, AC)]],
                              rlo_v.at[b], sems[b]).wait()
        pltpu.make_async_copy(nhi.at[idx_v.at[pl.ds(c * AC, AC)]],
                              rhi_v.at[b], sems[b]).wait()

    gi(0, 0)
    for c in range(ANC):
        b = c % 2
        if c + 1 < ANC:
            gi(c + 1, 1 - b)
        gw(c, b)
        pltpu.sync_copy(rlo_v.at[b], olo.at[pl.ds(base + c * AC, AC)])
        pltpu.sync_copy(rhi_v.at[b], ohi.at[pl.ds(base + c * AC, AC)])


def kernel(coords, features, edge_index, atom_counts, label_binary,
           W_in, W1, W2, W3, gamma, beta):
    xp = jnp.pad(features, ((0, NROWS - N), (0, 32 - IN_DIM)))
    wip = jnp.pad(W_in, ((0, 32 - IN_DIM), (0, 0)))
    src = edge_index[0]
    dst = edge_index[1]
    pad_e = EPAD - E
    srcp = jnp.concatenate([src, jnp.zeros((pad_e,), jnp.int32)])
    dstp = jnp.concatenate([dst, jnp.full((pad_e,), N, jnp.int32)])
    dlo2, dhi2 = _dmap(dstp)
    g2 = gamma.reshape(2, 16)
    b2 = beta.reshape(2, 16)

    zrows = jnp.zeros((ASTRIPE, 16), jnp.float32)
    hlo, hhi, glo, ghi = _h0(xp, wip, W1)
    for wn in (W2, W3, None):
        alo, ahi = _edge_sum(zrows, glo, ghi, srcp, dlo2, dhi2)
        if wn is None:
            hlo, hhi = _radd(hlo, hhi, alo, ahi)
        else:
            hlo, hhi, glo, ghi = _radd_pre(hlo, hhi, alo, ahi, wn)
    nlo, nhi = _bn(hlo, hhi, g2, b2)

    counts = atom_counts.astype(jnp.int32)
    offs = jnp.cumsum(counts) - counts
    slot = jnp.arange(MAX_ATOMS, dtype=jnp.int32)
    idx = offs[:, None] + slot[None, :]
    mask = slot[None, :] < counts[:, None]
    idxm = jnp.where(mask, jnp.clip(idx, 0, N - 1), N).reshape(-1)
    idxg = jnp.concatenate(
        [idxm, jnp.full((AT - R_RES * MAX_ATOMS,), N, jnp.int32)])

    olo, ohi = _final_gather(nlo, nhi, idxg)
    nat = R_RES * MAX_ATOMS
    aa = jnp.concatenate([olo[:nat], ohi[:nat]], axis=1).reshape(R_RES, MAX_ATOMS * 32)
    return (aa, label_binary)


# speedup vs baseline: 8.3468x; 1.0153x over previous
import functools

import jax
import jax.numpy as jnp
from jax import lax
from jax.experimental import pallas as pl
from jax.experimental.pallas import tpu as pltpu
from jax.experimental.pallas import tpu_sc as plsc

N = 100000
E = 1600000
R_RES = 7000
IN_DIM = 30
MAX_ATOMS = 14

NROWS = 100352
TH = 50176
ACC_ROWS = 53248
ASTRIPE = ACC_ROWS // 16
DR = TH // 16
CH = 2048
CPR = CH // 128
NCH = 49
TILE_E = CH * NCH
EPAD = 16 * TILE_E
BR = 2048
NB = NROWS // BR
AT = 98304
ATW = AT // 32
AC = 512
ANC = ATW // AC

_mesh = plsc.VectorSubcoreMesh(core_axis_name="c", subcore_axis_name="s")
_sc_params = pltpu.CompilerParams(use_tc_tiling_on_sc=False)


def _h0_body(x_ref, w_ref, w1_ref, lo_ref, hi_ref, glo_ref, ghi_ref):
    h = jnp.maximum(
        jnp.dot(x_ref[...], w_ref[...], preferred_element_type=jnp.float32), 0.0)
    g = jnp.dot(h, w1_ref[...], preferred_element_type=jnp.float32)
    lo_ref[...] = h[:, :16]
    hi_ref[...] = h[:, 16:]
    glo_ref[...] = g[:, :16]
    ghi_ref[...] = g[:, 16:]


def _h0(xp, wp, w1):
    return pl.pallas_call(
        _h0_body,
        grid=(NB,),
        in_specs=[
            pl.BlockSpec((BR, 32), lambda i: (i, 0)),
            pl.BlockSpec((32, 32), lambda i: (0, 0)),
            pl.BlockSpec((32, 32), lambda i: (0, 0)),
        ],
        out_specs=[pl.BlockSpec((BR, 16), lambda i: (i, 0))] * 4,
        out_shape=[jax.ShapeDtypeStruct((NROWS, 16), jnp.float32)] * 4,
    )(xp, wp, w1)


@functools.partial(
    pl.kernel,
    mesh=_mesh,
    out_type=[jax.ShapeDtypeStruct((NROWS, 16), jnp.float32)] * 2,
    scratch_types=[
        pltpu.VMEM((2, CH), jnp.int32),
        pltpu.VMEM((2, CPR, 128), jnp.int32),
        pltpu.VMEM((2, CH, 16), jnp.float32),
        pltpu.VMEM_SHARED((ACC_ROWS, 16), jnp.float32),
        pltpu.SemaphoreType.DMA,
        pltpu.SemaphoreType.DMA,
        pltpu.SemaphoreType.DMA,
    ],
    compiler_params=_sc_params,
)
def _edge_sum(zrows, hlo, hhi, srcp, dlo2, dhi2, alo, ahi,
              src_v, dst_m, rows_v, acc_sh, sem_g0, sem_g1, sem_s):
    cid = lax.axis_index("c")
    t = lax.axis_index("s")
    sem_g = (sem_g0, sem_g1)

    def one_pass(h_tab, d_tab, a_out, p):
        pltpu.sync_copy(zrows, acc_sh.at[pl.ds(t * ASTRIPE, ASTRIPE)])
        plsc.subcore_barrier()

        def idx_load(c, b):
            c0 = t * NCH + c
            pltpu.sync_copy(srcp.at[pl.ds(c0 * CH, CH)], src_v.at[b])
            pltpu.sync_copy(d_tab.at[pl.ds(c0 * CPR, CPR)], dst_m.at[b])

        def g_issue(b):
            pltpu.async_copy(h_tab.at[src_v.at[b]], rows_v.at[b], sem_g[b])

        def g_wait(b):
            pltpu.make_async_copy(h_tab.at[src_v.at[b]], rows_v.at[b],
                                  sem_g[b]).wait()

        def scat(b):
            for j in range(CPR):
                pltpu.async_copy(rows_v.at[b].at[pl.ds(j * 128, 128)],
                                 acc_sh.at[dst_m.at[b].at[j]], sem_s, add=True)
            for j in range(CPR):
                pltpu.make_async_copy(rows_v.at[b].at[pl.ds(j * 128, 128)],
                                      acc_sh.at[dst_m.at[b].at[j]], sem_s).wait()

        idx_load(0, 0)
        g_issue(0)

        def body(g2, carry):
            for k in (0, 1):
                c = 2 * g2 + k
                b, b1 = k, 1 - k

                @pl.when(c < NCH)
                def _():
                    @pl.when(c + 1 < NCH)
                    def _():
                        idx_load(c + 1, b1)

                    g_wait(b)

                    @pl.when(c + 1 < NCH)
                    def _():
                        g_issue(b1)

                    scat(b)

            return carry

        lax.fori_loop(0, (NCH + 1) // 2, body, 0)
        plsc.subcore_barrier()

        pltpu.sync_copy(acc_sh.at[pl.ds(t * DR, DR)],
                        a_out.at[pl.ds(p * TH + t * DR, DR)])
        plsc.subcore_barrier()

    def both_passes(h_tab, a_out):
        one_pass(h_tab, dlo2, a_out, 0)
        one_pass(h_tab, dhi2, a_out, 1)

    @pl.when(cid == 0)
    def _():
        both_passes(hlo, alo)

    @pl.when(cid == 1)
    def _():
        both_passes(hhi, ahi)


DW = EPAD // 32 // 128
DC = 49


@functools.partial(
    pl.kernel,
    mesh=_mesh,
    out_type=[jax.ShapeDtypeStruct((EPAD // 128, 128), jnp.int32)] * 2,
    scratch_types=[
        pltpu.VMEM((DC * 128,), jnp.int32),
        pltpu.VMEM((DC, 128), jnp.int32),
        pltpu.VMEM((DC, 128), jnp.int32),
    ],
    compiler_params=_sc_params,
)
def _dmap(dstp, dlo2, dhi2, din, olo_v, ohi_v):
    cid = lax.axis_index("c")
    sid = lax.axis_index("s")
    w = sid * 2 + cid

    def chunk(c, carry):
        r0 = w * DW + c * DC
        pltpu.sync_copy(dstp.at[pl.ds(r0 * 128, DC * 128)], din)

        def row(j, carry2):
            for l in range(8):
                d = din[pl.ds((j * 8 + l) * 16, 16)]
                garb = TH + jnp.bitwise_and(d, 2047)
                olo_v[j, pl.ds(l * 16, 16)] = jnp.where(d < TH, d, garb)
                rel = d - TH
                ohi_v[j, pl.ds(l * 16, 16)] = jnp.where(rel >= 0, rel, garb)
            return carry2

        lax.fori_loop(0, DC, row, 0)
        pltpu.sync_copy(olo_v, dlo2.at[pl.ds(r0, DC)])
        pltpu.sync_copy(ohi_v, dhi2.at[pl.ds(r0, DC)])
        return carry

    lax.fori_loop(0, DW // DC, chunk, 0)


def _radd_pre_body(lo_ref, hi_ref, alo_ref, ahi_ref, w_ref,
                   nlo_ref, nhi_ref, glo_ref, ghi_ref):
    i = pl.program_id(0)
    row = i * BR + lax.broadcasted_iota(jnp.int32, (BR, 16), 0)
    m = row < N
    nlo = jnp.where(m, lo_ref[...] + jnp.maximum(alo_ref[...], 0.0), 0.0)
    nhi = jnp.where(m, hi_ref[...] + jnp.maximum(ahi_ref[...], 0.0), 0.0)
    g = (jnp.dot(nlo, w_ref[:16, :], preferred_element_type=jnp.float32)
         + jnp.dot(nhi, w_ref[16:, :], preferred_element_type=jnp.float32))
    nlo_ref[...] = nlo
    nhi_ref[...] = nhi
    glo_ref[...] = g[:, :16]
    ghi_ref[...] = g[:, 16:]


def _radd_pre(hlo, hhi, alo, ahi, w):
    return pl.pallas_call(
        _radd_pre_body,
        grid=(NB,),
        in_specs=[pl.BlockSpec((BR, 16), lambda i: (i, 0))] * 4
        + [pl.BlockSpec((32, 32), lambda i: (0, 0))],
        out_specs=[pl.BlockSpec((BR, 16), lambda i: (i, 0))] * 4,
        out_shape=[jax.ShapeDtypeStruct((NROWS, 16), jnp.float32)] * 4,
    )(hlo, hhi, alo, ahi, w)


def _bn_body(lo_ref, hi_ref, alo_ref, ahi_ref, g_ref, b_ref,
             nlo_ref, nhi_ref, acc_ref):
    p = pl.program_id(0)
    j = pl.program_id(1)
    row = j * BR + lax.broadcasted_iota(jnp.int32, (BR, 16), 0)
    m = row < N
    lo = jnp.where(m, lo_ref[...] + jnp.maximum(alo_ref[...], 0.0), 0.0)
    hi = jnp.where(m, hi_ref[...] + jnp.maximum(ahi_ref[...], 0.0), 0.0)

    @pl.when(jnp.logical_and(p == 0, j == 0))
    def _():
        acc_ref[...] = jnp.zeros_like(acc_ref)

    @pl.when(p == 0)
    def _():
        acc_ref[0:1, :] = acc_ref[0:1, :] + jnp.sum(lo, axis=0, keepdims=True)
        acc_ref[1:2, :] = acc_ref[1:2, :] + jnp.sum(hi, axis=0, keepdims=True)
        acc_ref[2:3, :] = acc_ref[2:3, :] + jnp.sum(lo * lo, axis=0, keepdims=True)
        acc_ref[3:4, :] = acc_ref[3:4, :] + jnp.sum(hi * hi, axis=0, keepdims=True)

    @pl.when(p == 1)
    def _():
        inv_n = 1.0 / N
        mean_lo = acc_ref[0:1, :] * inv_n
        mean_hi = acc_ref[1:2, :] * inv_n
        var_lo = acc_ref[2:3, :] * inv_n - mean_lo * mean_lo
        var_hi = acc_ref[3:4, :] * inv_n - mean_hi * mean_hi
        s_lo = g_ref[0:1, :] * lax.rsqrt(var_lo + 1e-5)
        s_hi = g_ref[1:2, :] * lax.rsqrt(var_hi + 1e-5)
        y_lo = jnp.maximum((lo - mean_lo) * s_lo + b_ref[0:1, :], 0.0)
        y_hi = jnp.maximum((hi - mean_hi) * s_hi + b_ref[1:2, :], 0.0)
        nlo_ref[...] = jnp.where(m, y_lo, 0.0)
        nhi_ref[...] = jnp.where(m, y_hi, 0.0)


def _bn(hlo, hhi, alo, ahi, g2, b2):
    return pl.pallas_call(
        _bn_body,
        grid=(2, NB),
        in_specs=[pl.BlockSpec((BR, 16), lambda p, j: (j, 0))] * 4
        + [pl.BlockSpec((2, 16), lambda p, j: (0, 0))] * 2,
        out_specs=[pl.BlockSpec((BR, 16), lambda p, j: (j, 0))] * 2,
        out_shape=[jax.ShapeDtypeStruct((NROWS, 16), jnp.float32)] * 2,
        scratch_shapes=[pltpu.VMEM((8, 16), jnp.float32)],
    )(hlo, hhi, alo, ahi, g2, b2)


@functools.partial(
    pl.kernel,
    mesh=_mesh,
    out_type=[jax.ShapeDtypeStruct((AT, 16), jnp.float32)] * 2,
    scratch_types=[
        pltpu.VMEM((ATW,), jnp.int32),
        pltpu.VMEM((2, AC, 16), jnp.float32),
        pltpu.VMEM((2, AC, 16), jnp.float32),
        pltpu.SemaphoreType.DMA,
        pltpu.SemaphoreType.DMA,
    ],
    compiler_params=_sc_params,
)
def _final_gather(nlo, nhi, idxg, olo, ohi, idx_v, rlo_v, rhi_v, s0, s1):
    cid = lax.axis_index("c")
    sid = lax.axis_index("s")
    sems = (s0, s1)
    base = (sid * 2 + cid) * ATW
    pltpu.sync_copy(idxg.at[pl.ds(base, ATW)], idx_v)

    def gi(c, b):
        pltpu.async_copy(nlo.at[idx_v.at[pl.ds(c * AC, AC)]], rlo_v.at[b],
                         sems[b])
        pltpu.async_copy(nhi.at[idx_v.at[pl.ds(c * AC, AC)]], rhi_v.at[b],
                         sems[b])

    def gw(c, b):
        pltpu.make_async_copy(nlo.at[idx_v.at[pl.ds(c * AC, AC)]],
                              rlo_v.at[b], sems[b]).wait()
        pltpu.make_async_copy(nhi.at[idx_v.at[pl.ds(c * AC, AC)]],
                              rhi_v.at[b], sems[b]).wait()

    gi(0, 0)
    for c in range(ANC):
        b = c % 2
        if c + 1 < ANC:
            gi(c + 1, 1 - b)
        gw(c, b)
        pltpu.sync_copy(rlo_v.at[b], olo.at[pl.ds(base + c * AC, AC)])
        pltpu.sync_copy(rhi_v.at[b], ohi.at[pl.ds(base + c * AC, AC)])


def kernel(coords, features, edge_index, atom_counts, label_binary,
           W_in, W1, W2, W3, gamma, beta):
    xp = jnp.pad(features, ((0, NROWS - N), (0, 32 - IN_DIM)))
    wip = jnp.pad(W_in, ((0, 32 - IN_DIM), (0, 0)))
    src = edge_index[0]
    dst = edge_index[1]
    pad_e = EPAD - E
    srcp = jnp.concatenate([src, jnp.zeros((pad_e,), jnp.int32)])
    dstp = jnp.concatenate([dst, jnp.full((pad_e,), N, jnp.int32)])
    dlo2, dhi2 = _dmap(dstp)
    g2 = gamma.reshape(2, 16)
    b2 = beta.reshape(2, 16)

    zrows = jnp.zeros((ASTRIPE, 16), jnp.float32)
    hlo, hhi, glo, ghi = _h0(xp, wip, W1)
    for wn in (W2, W3, None):
        alo, ahi = _edge_sum(zrows, glo, ghi, srcp, dlo2, dhi2)
        if wn is not None:
            hlo, hhi, glo, ghi = _radd_pre(hlo, hhi, alo, ahi, wn)
    nlo, nhi = _bn(hlo, hhi, alo, ahi, g2, b2)

    counts = atom_counts.astype(jnp.int32)
    offs = jnp.cumsum(counts) - counts
    slot = jnp.arange(MAX_ATOMS, dtype=jnp.int32)
    idx = offs[:, None] + slot[None, :]
    mask = slot[None, :] < counts[:, None]
    idxm = jnp.where(mask, jnp.clip(idx, 0, N - 1), N).reshape(-1)
    idxg = jnp.concatenate(
        [idxm, jnp.full((AT - R_RES * MAX_ATOMS,), N, jnp.int32)])

    olo, ohi = _final_gather(nlo, nhi, idxg)
    nat = R_RES * MAX_ATOMS
    aa = jnp.concatenate([olo[:nat], ohi[:nat]], axis=1).reshape(R_RES, MAX_ATOMS * 32)
    return (aa, label_binary)
